# baseline (speedup 1.0000x reference)
import functools

import jax
import jax.numpy as jnp
from jax import lax
from jax.experimental import pallas as pl
from jax.experimental.pallas import tpu as pltpu
from jax.experimental.pallas import tpu_sc as plsc

D_MODEL = 128
NUM_WORKERS = 32
CHUNK = 320
NBUF = 3


@functools.partial(jax.jit, static_argnames=("b_per_w", "n_chunks"))
def _sc_gather(x_flat, table, b_per_w, n_chunks):
    mesh = plsc.VectorSubcoreMesh(core_axis_name="c", subcore_axis_name="s")
    total = x_flat.shape[0]

    @functools.partial(
        pl.kernel,
        out_type=jax.ShapeDtypeStruct((total, D_MODEL), jnp.float32),
        mesh=mesh,
        scratch_types=(
            [pltpu.VMEM((b_per_w,), jnp.int32)]
            + [pltpu.VMEM((CHUNK, D_MODEL), jnp.float32) for _ in range(NBUF)]
            + [pltpu.SemaphoreType.DMA for _ in range(2 * NBUF)]
        ),
    )
    def k(x_hbm, tbl_hbm, out_hbm, idx_v, *bufs_and_sems):
        bufs = bufs_and_sems[:NBUF]
        gsems = bufs_and_sems[NBUF:2 * NBUF]
        osems = bufs_and_sems[2 * NBUF:]
        wid = lax.axis_index("s") * 2 + lax.axis_index("c")
        base = wid * b_per_w
        pltpu.sync_copy(x_hbm.at[pl.ds(base, b_per_w)], idx_v)

        def gather_start(c, b):
            return pltpu.async_copy(
                tbl_hbm.at[idx_v.at[pl.ds(c * CHUNK, CHUNK)]], bufs[b], gsems[b]
            )

        def out_start(c, b):
            return pltpu.async_copy(
                bufs[b], out_hbm.at[pl.ds(base + c * CHUNK, CHUNK)], osems[b]
            )

        gcp = [None] * NBUF
        ocp = [None] * NBUF
        for b in range(min(NBUF, n_chunks)):
            gcp[b] = gather_start(b, b)
        for i in range(n_chunks):
            b = i % NBUF
            gcp[b].wait()
            ocp[b] = out_start(i, b)
            nxt = i + NBUF
            if nxt < n_chunks:
                ocp[b].wait()
                ocp[b] = None
                gcp[b] = gather_start(nxt, b)
        for cp in ocp:
            if cp is not None:
                cp.wait()

    return k(x_flat, table)


def kernel(x, table):
    n, s = x.shape
    total = n * s
    b_per_w = total // NUM_WORKERS
    n_chunks = b_per_w // CHUNK
    xt_flat = jnp.transpose(x).reshape(total).astype(jnp.int32)
    out = _sc_gather(xt_flat, table, b_per_w, n_chunks)
    return jnp.transpose(out.reshape(s, n, D_MODEL), (1, 0, 2))
